# baseline (speedup 1.0000x reference)
import functools

import jax
import jax.numpy as jnp
from jax import lax
from jax.experimental import pallas as pl
from jax.experimental.pallas import tpu as pltpu
from jax.experimental.pallas import tpu_sc as plsc

_BATCH = 16384
_SEQ = 200
_DIM = 16
_N_ROWS = _BATCH * _SEQ
_NUM_WORKERS = 32
_ROWS_PER_WORKER = _N_ROWS // _NUM_WORKERS
_CHUNK = 2048
_CHUNKS = _ROWS_PER_WORKER // _CHUNK


def _sc_lookup(idx_hbm, table_hbm, out_hbm, idx_v, rows_v, sem):
    wid = lax.axis_index("s") * 2 + lax.axis_index("c")
    base = wid * _ROWS_PER_WORKER

    def body(c, carry):
        r0 = base + c * _CHUNK
        pltpu.sync_copy(idx_hbm.at[pl.ds(r0, _CHUNK)], idx_v)
        pltpu.async_copy(table_hbm.at[idx_v], rows_v, sem).wait()
        pltpu.sync_copy(rows_v, out_hbm.at[pl.ds(r0, _CHUNK)])
        return carry

    lax.fori_loop(0, _CHUNKS, body, 0)


def kernel(indices, table):
    idx = indices.reshape(_N_ROWS)
    if idx.dtype != jnp.int32:
        idx = idx.astype(jnp.int32)
    mesh = plsc.VectorSubcoreMesh(core_axis_name="c", subcore_axis_name="s")
    run = functools.partial(
        pl.kernel,
        out_type=jax.ShapeDtypeStruct((_N_ROWS, _DIM), jnp.float32),
        mesh=mesh,
        scratch_types=[
            pltpu.VMEM((_CHUNK,), jnp.int32),
            pltpu.VMEM((_CHUNK, _DIM), jnp.float32),
            pltpu.SemaphoreType.DMA,
        ],
        compiler_params=pltpu.CompilerParams(use_tc_tiling_on_sc=False),
    )(_sc_lookup)
    out = run(idx, table)
    return out.reshape(_BATCH, _SEQ, _DIM)


# speedup vs baseline: 48.5736x; 48.5736x over previous
import functools

import jax
import jax.numpy as jnp
from jax import lax
from jax.experimental import pallas as pl
from jax.experimental.pallas import tpu as pltpu
from jax.experimental.pallas import tpu_sc as plsc

_BATCH = 16384
_SEQ = 200
_DIM = 16
_N_ROWS = _BATCH * _SEQ
_NUM_WORKERS = 32
_UNITS = _SEQ * 8
_UNITS_PER_WORKER = _UNITS // _NUM_WORKERS
_UNIT_WORDS = 32768
_IDX_WORDS = 4096


def _sc_lookup(idxt_hbm, tflat_hbm, out_hbm, idx0, idx1, tflat_v, st0, st1,
               sem0, sem1, semi0, semi1):
    wid = lax.axis_index("s") * 2 + lax.axis_index("c")
    u0 = wid * _UNITS_PER_WORKER

    pltpu.sync_copy(tflat_hbm, tflat_v)

    idx = (idx0, idx1)
    st = (st0, st1)
    sems = (sem0, sem1)
    semi = (semi0, semi1)

    def idx_slice(u):
        su = u >> 3
        h = u & 3
        return idxt_hbm.at[pl.ds(su * _BATCH + h * _IDX_WORDS, _IDX_WORDS)]

    def start_idx(u, buf):
        pltpu.async_copy(idx_slice(u), idx[buf], semi[buf])

    def compute_unit(u, buf):
        te8 = ((u >> 2) & 1) << 3
        pltpu.make_async_copy(idx_slice(u0), idx[buf], semi[buf]).wait()

        @plsc.parallel_loop(0, 32)
        def tb(tb_l):
            for g in range(8):
                idxv = idx[buf][pl.ds(tb_l * 128 + g * 16, 16)]
                idxm = idxv * _DIM + te8
                for ei in range(8):
                    vals = plsc.load_gather(tflat_v, [idxm + ei])
                    st[buf][pl.ds(tb_l * 1024 + ei * 128 + g * 16, 16)] = vals

    def out_slice(u):
        return out_hbm.at[pl.ds(u * _UNIT_WORDS, _UNIT_WORDS)]

    start_idx(u0, 0)
    start_idx(u0 + 1, 1)
    for b in range(2):
        compute_unit(u0 + b, b)
        start_idx(u0 + b + 2, b)
        pltpu.async_copy(st[b], out_slice(u0 + b), sems[b])

    def round_body(r, carry):
        j0 = 2 * r
        for b in range(2):
            j = j0 + b
            pltpu.make_async_copy(st[b], out_slice(u0), sems[b]).wait()
            compute_unit(u0 + j, b)

            @pl.when(j + 2 < _UNITS_PER_WORKER)
            def _():
                start_idx(u0 + j + 2, b)

            pltpu.async_copy(st[b], out_slice(u0 + j), sems[b])
        return carry

    lax.fori_loop(1, _UNITS_PER_WORKER // 2, round_body, 0)

    for b in range(2):
        pltpu.make_async_copy(st[b], out_slice(u0), sems[b]).wait()


def kernel(indices, table):
    if indices.dtype != jnp.int32:
        indices = indices.astype(jnp.int32)
    idxt = jnp.swapaxes(indices, 0, 1).reshape(_N_ROWS)
    tflat = table.reshape(8 * _DIM)
    mesh = plsc.VectorSubcoreMesh(core_axis_name="c", subcore_axis_name="s")
    run = functools.partial(
        pl.kernel,
        out_type=jax.ShapeDtypeStruct((_N_ROWS * _DIM,), jnp.float32),
        mesh=mesh,
        scratch_types=[
            pltpu.VMEM((_IDX_WORDS,), jnp.int32),
            pltpu.VMEM((_IDX_WORDS,), jnp.int32),
            pltpu.VMEM((8 * _DIM,), jnp.float32),
            pltpu.VMEM((_UNIT_WORDS,), jnp.float32),
            pltpu.VMEM((_UNIT_WORDS,), jnp.float32),
            pltpu.SemaphoreType.DMA,
            pltpu.SemaphoreType.DMA,
            pltpu.SemaphoreType.DMA,
            pltpu.SemaphoreType.DMA,
        ],
        compiler_params=pltpu.CompilerParams(needs_layout_passes=False),
    )(_sc_lookup)
    out = run(idxt, tflat)
    out5 = out.reshape(_SEQ, 2, 128, 8, 128)
    return out5.transpose(2, 4, 0, 1, 3).reshape(_BATCH, _SEQ, _DIM)
